# baseline (speedup 1.0000x reference)
import jax
import jax.numpy as jnp
from jax import lax
from jax.experimental import pallas as pl
from jax.experimental.pallas import tpu as pltpu
from jax.experimental.pallas import tpu_sc as plsc

D_MODEL = 1024
BATCH = 16384
LANES = 16
NUM_CORES = 2
NUM_SUBCORES = 16

TC_ROWS = 8192
TC_BLOCK = 1024


def _tc_body(idx1_ref, idx2_ref, src_ref, dst_ref, out_ref):
    i1 = idx1_ref[0, 0]
    i2 = idx2_ref[0, 0]
    iota = lax.broadcasted_iota(jnp.int32, (TC_BLOCK, 64), 1)
    oh1 = (iota == i1[:, None]).astype(jnp.float32)
    oh2 = (iota == i2[:, None]).astype(jnp.float32)
    out_ref[...] = (
        jnp.dot(oh1, src_ref[...], preferred_element_type=jnp.float32)
        + jnp.dot(oh2, dst_ref[...], preferred_element_type=jnp.float32))


def _tc_run(src_w, dst_w, mov1, mov2):
    grid = TC_ROWS // TC_BLOCK
    return pl.pallas_call(
        _tc_body,
        grid=(grid,),
        in_specs=[
            pl.BlockSpec((1, 1, TC_BLOCK), lambda i: (i, 0, 0)),
            pl.BlockSpec((1, 1, TC_BLOCK), lambda i: (i, 0, 0)),
            pl.BlockSpec((64, D_MODEL), lambda i: (0, 0)),
            pl.BlockSpec((64, D_MODEL), lambda i: (0, 0)),
        ],
        out_specs=pl.BlockSpec((TC_BLOCK, D_MODEL), lambda i: (i, 0)),
        out_shape=jax.ShapeDtypeStruct((TC_ROWS, D_MODEL), jnp.float32),
    )(mov1.reshape(grid, 1, TC_BLOCK), mov2.reshape(grid, 1, TC_BLOCK),
      src_w, dst_w)


SC_ROWS = BATCH - TC_ROWS
BG = SC_ROWS // NUM_SUBCORES
DH = D_MODEL // NUM_CORES
CHUNK = 32
NRING = 4
LOOKAHEAD = 2
NUM_CHUNKS = BG // CHUNK
NGROUP = NUM_CHUNKS // NRING
NVREG = DH // LANES
REP_ROWS = NUM_CORES * NUM_SUBCORES * 64


def _sc_kernel(src_w, dst_w, mov1, mov2, out, rep, t1, idx1_v, idx2_v,
               b0, b1, b2, b3, g0, g1, g2, g3, o0, o1, o2, o3):
    bufs = (b0, b1, b2, b3)
    gsems = (g0, g1, g2, g3)
    osems = (o0, o1, o2, o3)
    s = lax.axis_index("s")
    c = lax.axis_index("c")
    row_base = s * BG
    col = c * DH

    pltpu.sync_copy(src_w.at[:, pl.ds(col, DH)], t1)
    pltpu.sync_copy(mov1.at[pl.ds(row_base, BG)], idx1_v.at[pl.ds(0, BG)])
    pltpu.sync_copy(mov2.at[pl.ds(row_base, BG)], idx2_v)

    rep_base = (c * NUM_SUBCORES + s) * 64
    for half in range(2):
        pltpu.sync_copy(dst_w.at[pl.ds(half * 32, 32), pl.ds(col, DH)], b0)
        pltpu.sync_copy(b0, rep.at[pl.ds(rep_base + half * 32, 32)])

    lane_block = c * (NUM_SUBCORES * 64) + lax.iota(jnp.int32, LANES) * 64

    def transform(v, carry):
        sl = pl.ds(v * LANES, LANES)
        idx2_v[sl] = idx2_v[sl] + lane_block
        return carry

    lax.fori_loop(0, BG // LANES, transform, 0)
    plsc.subcore_barrier()

    def gather(k, buf, sem):
        off = pl.multiple_of(k * CHUNK, CHUNK)
        pltpu.async_copy(rep.at[idx2_v.at[pl.ds(off, CHUNK)]], buf, sem)

    def wait_gather(k, buf, sem):
        off = pl.multiple_of(k * CHUNK, CHUNK)
        pltpu.make_async_copy(rep.at[idx2_v.at[pl.ds(off, CHUNK)]], buf,
                              sem).wait()

    def out_slice(k):
        return out.at[pl.ds(row_base + k * CHUNK, CHUNK), pl.ds(col, DH)]

    def add_rows(k, buf):
        @plsc.parallel_loop(0, CHUNK, unroll=2)
        def row_body(i):
            r1 = idx1_v[pl.ds(k * CHUNK + i, LANES)][0]
            for j in range(NVREG):
                sl = pl.ds(j * LANES, LANES)
                plsc.addupdate(buf.at[i, sl], t1[r1, sl])

    gather(0, bufs[0], gsems[0])
    gather(1, bufs[1], gsems[1])

    def group_body(g, carry):
        for p in range(NRING):
            k = g * NRING + p
            kg = k + LOOKAHEAD
            q = (p + LOOKAHEAD) % NRING

            def prep(_):
                def drain(_):
                    pltpu.make_async_copy(bufs[q], out_slice(kg - NRING),
                                          osems[q]).wait()
                    return 0

                lax.cond(kg - NRING >= 0, drain, lambda _: 0, 0)
                gather(kg, bufs[q], gsems[q])
                return 0

            lax.cond(kg < NUM_CHUNKS, prep, lambda _: 0, 0)
            wait_gather(k, bufs[p], gsems[p])
            add_rows(k, bufs[p])
            pltpu.async_copy(bufs[p], out_slice(k), osems[p])
        return carry

    lax.fori_loop(0, NGROUP, group_body, 0)
    for p in range(NRING):
        k = NUM_CHUNKS - NRING + p
        pltpu.make_async_copy(bufs[p], out_slice(k), osems[p]).wait()


def _sc_run(src_w, dst_w, mov1, mov2):
    kern = pl.kernel(
        _sc_kernel,
        mesh=plsc.VectorSubcoreMesh(core_axis_name="c", subcore_axis_name="s"),
        out_type=jax.ShapeDtypeStruct((SC_ROWS, D_MODEL), jnp.float32),
        scratch_types=[
            pltpu.HBM((REP_ROWS, DH), jnp.float32),
            pltpu.VMEM((64, DH), jnp.float32),
            pltpu.VMEM((BG + LANES,), jnp.int32),
            pltpu.VMEM((BG,), jnp.int32),
            pltpu.VMEM((CHUNK, DH), jnp.float32),
            pltpu.VMEM((CHUNK, DH), jnp.float32),
            pltpu.VMEM((CHUNK, DH), jnp.float32),
            pltpu.VMEM((CHUNK, DH), jnp.float32),
            pltpu.SemaphoreType.DMA,
            pltpu.SemaphoreType.DMA,
            pltpu.SemaphoreType.DMA,
            pltpu.SemaphoreType.DMA,
            pltpu.SemaphoreType.DMA,
            pltpu.SemaphoreType.DMA,
            pltpu.SemaphoreType.DMA,
            pltpu.SemaphoreType.DMA,
        ],
    )
    return kern(src_w, dst_w, mov1, mov2)


@jax.jit
def _run(src_w, dst_w, mov1, mov2):
    sc_out = _sc_run(src_w, dst_w, mov1[TC_ROWS:], mov2[TC_ROWS:])
    tc_out = _tc_run(src_w, dst_w, mov1[:TC_ROWS], mov2[:TC_ROWS])
    return lax.concatenate([tc_out, sc_out], 0)


def kernel(pieces, mov1, mov2, mov_src_w, mov_dst_w):
    del pieces
    return _run(mov_src_w, mov_dst_w, mov1, mov2)


# speedup vs baseline: 1.0424x; 1.0424x over previous
import jax
import jax.numpy as jnp
from jax import lax
from jax.experimental import pallas as pl
from jax.experimental.pallas import tpu as pltpu
from jax.experimental.pallas import tpu_sc as plsc

D_MODEL = 1024
BATCH = 16384
LANES = 16
NUM_CORES = 2
NUM_SUBCORES = 16
BG = BATCH // NUM_SUBCORES
DH = D_MODEL // NUM_CORES
CHUNK = 32
NRING = 4
LOOKAHEAD = 2
NUM_CHUNKS = BG // CHUNK
NGROUP = NUM_CHUNKS // NRING
NVREG = DH // LANES
REP_ROWS = NUM_CORES * NUM_SUBCORES * 64


def _sc_kernel(src_w, dst_w, mov1, mov2, out, rep, t1, idx1_v, idx2_v,
               b0, b1, b2, b3, g0, g1, g2, g3, o0, o1, o2, o3):
    bufs = (b0, b1, b2, b3)
    gsems = (g0, g1, g2, g3)
    osems = (o0, o1, o2, o3)
    s = lax.axis_index("s")
    c = lax.axis_index("c")
    row_base = s * BG
    col = c * DH

    pltpu.sync_copy(src_w.at[:, pl.ds(col, DH)], t1)
    pltpu.sync_copy(mov1.at[pl.ds(row_base, BG)], idx1_v.at[pl.ds(0, BG)])
    pltpu.sync_copy(mov2.at[pl.ds(row_base, BG)], idx2_v)

    rep_base = (c * NUM_SUBCORES + s) * 64
    for half in range(2):
        pltpu.sync_copy(dst_w.at[pl.ds(half * 32, 32), pl.ds(col, DH)], b0)
        pltpu.sync_copy(b0, rep.at[pl.ds(rep_base + half * 32, 32)])

    lane_block = c * (NUM_SUBCORES * 64) + lax.iota(jnp.int32, LANES) * 64

    def transform(v, carry):
        sl = pl.ds(v * LANES, LANES)
        idx2_v[sl] = idx2_v[sl] + lane_block
        return carry

    lax.fori_loop(0, BG // LANES, transform, 0)
    plsc.subcore_barrier()

    def gather(k, buf, sem):
        off = pl.multiple_of(k * CHUNK, CHUNK)
        pltpu.async_copy(rep.at[idx2_v.at[pl.ds(off, CHUNK)]], buf, sem)

    def wait_gather(k, buf, sem):
        off = pl.multiple_of(k * CHUNK, CHUNK)
        pltpu.make_async_copy(rep.at[idx2_v.at[pl.ds(off, CHUNK)]], buf,
                              sem).wait()

    def out_slice(k):
        return out.at[pl.ds(row_base + k * CHUNK, CHUNK), pl.ds(col, DH)]

    def add_rows(k, buf):
        @plsc.parallel_loop(0, CHUNK, unroll=4)
        def row_body(i):
            r1 = idx1_v[pl.ds(k * CHUNK + i, LANES)][0]
            for j in range(NVREG):
                sl = pl.ds(j * LANES, LANES)
                plsc.addupdate(buf.at[i, sl], t1[r1, sl])

    gather(0, bufs[0], gsems[0])
    gather(1, bufs[1], gsems[1])

    def group_body(g, carry):
        for p in range(NRING):
            k = g * NRING + p
            kg = k + LOOKAHEAD
            q = (p + LOOKAHEAD) % NRING

            def prep(_):
                def drain(_):
                    pltpu.make_async_copy(bufs[q], out_slice(kg - NRING),
                                          osems[q]).wait()
                    return 0

                lax.cond(kg - NRING >= 0, drain, lambda _: 0, 0)
                gather(kg, bufs[q], gsems[q])
                return 0

            lax.cond(kg < NUM_CHUNKS, prep, lambda _: 0, 0)
            wait_gather(k, bufs[p], gsems[p])
            add_rows(k, bufs[p])
            pltpu.async_copy(bufs[p], out_slice(k), osems[p])
        return carry

    lax.fori_loop(0, NGROUP, group_body, 0)
    for p in range(NRING):
        k = NUM_CHUNKS - NRING + p
        pltpu.make_async_copy(bufs[p], out_slice(k), osems[p]).wait()


@jax.jit
def _run(src_w, dst_w, mov1, mov2):
    kern = pl.kernel(
        _sc_kernel,
        mesh=plsc.VectorSubcoreMesh(core_axis_name="c", subcore_axis_name="s"),
        out_type=jax.ShapeDtypeStruct((BATCH, D_MODEL), jnp.float32),
        scratch_types=[
            pltpu.HBM((REP_ROWS, DH), jnp.float32),
            pltpu.VMEM((64, DH), jnp.float32),
            pltpu.VMEM((BG + LANES,), jnp.int32),
            pltpu.VMEM((BG,), jnp.int32),
            pltpu.VMEM((CHUNK, DH), jnp.float32),
            pltpu.VMEM((CHUNK, DH), jnp.float32),
            pltpu.VMEM((CHUNK, DH), jnp.float32),
            pltpu.VMEM((CHUNK, DH), jnp.float32),
            pltpu.SemaphoreType.DMA,
            pltpu.SemaphoreType.DMA,
            pltpu.SemaphoreType.DMA,
            pltpu.SemaphoreType.DMA,
            pltpu.SemaphoreType.DMA,
            pltpu.SemaphoreType.DMA,
            pltpu.SemaphoreType.DMA,
            pltpu.SemaphoreType.DMA,
        ],
    )
    return kern(src_w, dst_w, mov1, mov2)


def kernel(pieces, mov1, mov2, mov_src_w, mov_dst_w):
    del pieces
    return _run(mov_src_w, mov_dst_w, mov1, mov2)


# speedup vs baseline: 1.3109x; 1.2576x over previous
import jax
import jax.numpy as jnp
from jax import lax
from jax.experimental import pallas as pl
from jax.experimental.pallas import tpu as pltpu
from jax.experimental.pallas import tpu_sc as plsc

D_MODEL = 1024
BATCH = 16384
LANES = 16
NUM_CORES = 2
NUM_SUBCORES = 16
BG = BATCH // NUM_SUBCORES
DH = D_MODEL // NUM_CORES
CHUNK = 32
NRING = 4
LOOKAHEAD = 2
NUM_CHUNKS = BG // CHUNK
NGROUP = NUM_CHUNKS // NRING
NVREG = DH // LANES
REP_ROWS = NUM_CORES * NUM_SUBCORES * 64


def _sc_kernel(src_w, dst_w, mov1, mov2, out, rep, t1, idx1_v, idx2_v,
               b0, b1, b2, b3, g0, g1, g2, g3, o0, o1, o2, o3):
    bufs = (b0, b1, b2, b3)
    gsems = (g0, g1, g2, g3)
    osems = (o0, o1, o2, o3)
    s = lax.axis_index("s")
    c = lax.axis_index("c")
    row_base = s * BG
    col = c * DH

    pltpu.sync_copy(src_w.at[:, pl.ds(col, DH)], t1)
    pltpu.sync_copy(mov1.at[pl.ds(row_base, BG)], idx1_v.at[pl.ds(0, BG)])
    pltpu.sync_copy(mov2.at[pl.ds(row_base, BG)], idx2_v)

    rep_base = (c * NUM_SUBCORES + s) * 64
    for half in range(2):
        pltpu.sync_copy(dst_w.at[pl.ds(half * 32, 32), pl.ds(col, DH)], b0)
        pltpu.sync_copy(b0, rep.at[pl.ds(rep_base + half * 32, 32)])

    lane_block = c * (NUM_SUBCORES * 64) + lax.iota(jnp.int32, LANES) * 64

    def transform(v, carry):
        sl = pl.ds(v * LANES, LANES)
        idx2_v[sl] = idx2_v[sl] + lane_block
        return carry

    lax.fori_loop(0, BG // LANES, transform, 0)
    plsc.subcore_barrier()

    def gather(k, buf, sem):
        off = pl.multiple_of(k * CHUNK, CHUNK)
        pltpu.async_copy(rep.at[idx2_v.at[pl.ds(off, CHUNK)]], buf, sem)

    def wait_gather(k, buf, sem):
        off = pl.multiple_of(k * CHUNK, CHUNK)
        pltpu.make_async_copy(rep.at[idx2_v.at[pl.ds(off, CHUNK)]], buf,
                              sem).wait()

    def out_slice(k):
        return out.at[pl.ds(row_base + k * CHUNK, CHUNK), pl.ds(col, DH)]

    def add_rows(k, buf):
        @plsc.parallel_loop(0, CHUNK, unroll=1)
        def row_body(i):
            r1 = idx1_v[pl.ds(k * CHUNK + i, LANES)][0]
            for j in range(NVREG):
                sl = pl.ds(j * LANES, LANES)
                plsc.addupdate(buf.at[i, sl], t1[r1, sl])

    gather(0, bufs[0], gsems[0])
    gather(1, bufs[1], gsems[1])

    def group_body(g, carry):
        for p in range(NRING):
            k = g * NRING + p
            kg = k + LOOKAHEAD
            q = (p + LOOKAHEAD) % NRING

            def prep(_):
                def drain(_):
                    pltpu.make_async_copy(bufs[q], out_slice(kg - NRING),
                                          osems[q]).wait()
                    return 0

                lax.cond(kg - NRING >= 0, drain, lambda _: 0, 0)
                gather(kg, bufs[q], gsems[q])
                return 0

            lax.cond(kg < NUM_CHUNKS, prep, lambda _: 0, 0)
            wait_gather(k, bufs[p], gsems[p])
            add_rows(k, bufs[p])
            pltpu.async_copy(bufs[p], out_slice(k), osems[p])
        return carry

    lax.fori_loop(0, NGROUP, group_body, 0)
    for p in range(NRING):
        k = NUM_CHUNKS - NRING + p
        pltpu.make_async_copy(bufs[p], out_slice(k), osems[p]).wait()


@jax.jit
def _run(src_w, dst_w, mov1, mov2):
    kern = pl.kernel(
        _sc_kernel,
        mesh=plsc.VectorSubcoreMesh(core_axis_name="c", subcore_axis_name="s"),
        out_type=jax.ShapeDtypeStruct((BATCH, D_MODEL), jnp.float32),
        scratch_types=[
            pltpu.HBM((REP_ROWS, DH), jnp.float32),
            pltpu.VMEM((64, DH), jnp.float32),
            pltpu.VMEM((BG + LANES,), jnp.int32),
            pltpu.VMEM((BG,), jnp.int32),
            pltpu.VMEM((CHUNK, DH), jnp.float32),
            pltpu.VMEM((CHUNK, DH), jnp.float32),
            pltpu.VMEM((CHUNK, DH), jnp.float32),
            pltpu.VMEM((CHUNK, DH), jnp.float32),
            pltpu.SemaphoreType.DMA,
            pltpu.SemaphoreType.DMA,
            pltpu.SemaphoreType.DMA,
            pltpu.SemaphoreType.DMA,
            pltpu.SemaphoreType.DMA,
            pltpu.SemaphoreType.DMA,
            pltpu.SemaphoreType.DMA,
            pltpu.SemaphoreType.DMA,
        ],
    )
    return kern(src_w, dst_w, mov1, mov2)


def kernel(pieces, mov1, mov2, mov_src_w, mov_dst_w):
    del pieces
    return _run(mov_src_w, mov_dst_w, mov1, mov2)
